# Initial kernel scaffold; baseline (speedup 1.0000x reference)
import jax
import jax.numpy as jnp
from jax.experimental import pallas as pl


def kernel(x, g_fc1_w, g_fc1_b, g_fc1_gamma, g_fc1_beta, mr_w, mr_b, mr_gamma, mr_beta, g_fc2_w, g_fc2_b, g_fc2_gamma, g_fc2_beta, f_fc1_w, f_fc1_b, f_fc1_gamma, f_fc1_beta, f_fc2_w, f_fc2_b, f_fc2_gamma, f_fc2_beta, rel_pos):
    raise NotImplementedError("write your pallas kernel here")


# speedup vs baseline: 271.5349x; 271.5349x over previous
import numpy as np
import jax
import jax.numpy as jnp
from jax.experimental import pallas as pl
from jax.experimental.pallas import tpu as pltpu

_C = 96
_K = 9
_G = 4
_EPS = 1e-05
_B, _H, _W = 16, 32, 32
_N = _H * _W
_R = 256
_PREC = jax.lax.Precision.HIGHEST


def _gelu(x):
    return 0.5 * x * (1.0 + jax.lax.erf(x * np.float32(2.0 ** -0.5)))


def _body(xt_full_ref, xt_tile_ref, rp_ref,
          w1_ref, s1_ref, t1_ref,
          a1_ref, a2_ref, s2_ref, t2_ref,
          g2_ref, s3_ref, t3_ref,
          f1_ref, s4_ref, t4_ref,
          f2_ref, s5_ref, t5_ref,
          out_ref):
    xt = xt_full_ref[0]
    xt_t = xt_tile_ref[0]
    rp = rp_ref[0]

    h = jnp.dot(xt, w1_ref[...], preferred_element_type=jnp.float32,
                precision=_PREC) * s1_ref[...] + t1_ref[...]
    nrm = jnp.sqrt(jnp.sum(h * h, axis=1, keepdims=True))
    xn = h / jnp.maximum(nrm, 1e-12)
    xn2 = xn * xn
    ones_row = jnp.ones((1, _C), jnp.float32)
    sq_row = jax.lax.dot_general(ones_row, xn2, (((1,), (1,)), ((), ())),
                                 preferred_element_type=jnp.float32,
                                 precision=_PREC)

    h_t = jnp.dot(xt_t, w1_ref[...], preferred_element_type=jnp.float32,
                  precision=_PREC) * s1_ref[...] + t1_ref[...]
    nrm_t = jnp.sqrt(jnp.sum(h_t * h_t, axis=1, keepdims=True))
    xn_t = h_t / jnp.maximum(nrm_t, 1e-12)
    sq_t = jnp.sum(xn_t * xn_t, axis=1, keepdims=True)

    inner = jax.lax.dot_general(xn_t, xn, (((1,), (1,)), ((), ())),
                                preferred_element_type=jnp.float32,
                                precision=_PREC)
    d = sq_t + (-2.0) * inner + sq_row + rp

    col = jax.lax.broadcasted_iota(jnp.int32, (_R, _N), 1)
    maxg = jnp.full((_R, _C), -1e30, jnp.float32)
    for _ in range(_K):
        m = jnp.min(d, axis=1, keepdims=True)
        idx = jnp.min(jnp.where(d == m, col, _N), axis=1, keepdims=True)
        sel = col == idx
        onehot = sel.astype(jnp.float32)
        gk = jnp.dot(onehot, h, preferred_element_type=jnp.float32,
                     precision=_PREC)
        maxg = jnp.maximum(maxg, gk)
        d = jnp.where(sel, 1e30, d)
    xjm = maxg - h_t

    mr = (jnp.dot(h_t, a1_ref[...], preferred_element_type=jnp.float32,
                  precision=_PREC)
          + jnp.dot(xjm, a2_ref[...], preferred_element_type=jnp.float32,
                    precision=_PREC))
    mr = _gelu(mr * s2_ref[...] + t2_ref[...])

    g = jnp.dot(mr, g2_ref[...], preferred_element_type=jnp.float32,
                precision=_PREC) * s3_ref[...] + t3_ref[...]
    score = g + xt_t

    f = _gelu(jnp.dot(score, f1_ref[...], preferred_element_type=jnp.float32,
                      precision=_PREC) * s4_ref[...] + t4_ref[...])
    f = jnp.dot(f, f2_ref[...], preferred_element_type=jnp.float32,
                precision=_PREC) * s5_ref[...] + t5_ref[...]
    out_ref[0] = f + score


def kernel(x, g_fc1_w, g_fc1_b, g_fc1_gamma, g_fc1_beta, mr_w, mr_b,
           mr_gamma, mr_beta, g_fc2_w, g_fc2_b, g_fc2_gamma, g_fc2_beta,
           f_fc1_w, f_fc1_b, f_fc1_gamma, f_fc1_beta, f_fc2_w, f_fc2_b,
           f_fc2_gamma, f_fc2_beta, rel_pos):
    inv = np.float32(1.0 / np.sqrt(1.0 + _EPS))

    def fold(w_b, gamma, beta):
        s = gamma * inv
        return s[None, :], (w_b * s + beta)[None, :]

    s1, t1 = fold(g_fc1_b, g_fc1_gamma, g_fc1_beta)
    s2i, t2i = fold(mr_b, mr_gamma, mr_beta)
    s3, t3 = fold(g_fc2_b, g_fc2_gamma, g_fc2_beta)
    s4, t4 = fold(f_fc1_b, f_fc1_gamma, f_fc1_beta)
    s5, t5 = fold(f_fc2_b, f_fc2_gamma, f_fc2_beta)

    q = np.concatenate([2 * np.arange(_C), 2 * np.arange(_C) + 1])
    wg = mr_w.reshape(_G, 2 * _C // _G, 2 * _C // _G)
    w2 = jax.scipy.linalg.block_diag(*[wg[g] for g in range(_G)])
    a = w2[q][:, q]
    a1 = a[:, :_C].T
    a2 = a[:, _C:].T
    s2 = s2i[:, q]
    t2 = t2i[:, q]
    g2 = g_fc2_w[:, q].T

    w1 = g_fc1_w.T
    f1 = f_fc1_w.T
    f2 = f_fc2_w.T

    xt = jnp.transpose(x.reshape(_B, _C, _N), (0, 2, 1))

    nt = _N // _R
    grid = (nt, _B)

    def full_img(t, b):
        return (b, 0, 0)

    def row_tile(t, b):
        return (b, t, 0)

    def rp_tile(t, b):
        return (0, t, 0)

    def w_map(t, b):
        return (0, 0)

    wspec = lambda shape: pl.BlockSpec(shape, w_map)

    out = pl.pallas_call(
        _body,
        grid=grid,
        in_specs=[
            pl.BlockSpec((1, _N, _C), full_img),
            pl.BlockSpec((1, _R, _C), row_tile),
            pl.BlockSpec((1, _R, _N), rp_tile),
            wspec((_C, _C)),
            wspec((1, _C)), wspec((1, _C)),
            wspec((_C, 2 * _C)), wspec((_C, 2 * _C)),
            wspec((1, 2 * _C)), wspec((1, 2 * _C)),
            wspec((2 * _C, _C)),
            wspec((1, _C)), wspec((1, _C)),
            wspec((_C, 4 * _C)),
            wspec((1, 4 * _C)), wspec((1, 4 * _C)),
            wspec((4 * _C, _C)),
            wspec((1, _C)), wspec((1, _C)),
        ],
        out_specs=pl.BlockSpec((1, _R, _C), row_tile),
        out_shape=jax.ShapeDtypeStruct((_B, _N, _C), jnp.float32),
        compiler_params=pltpu.CompilerParams(
            dimension_semantics=("arbitrary", "arbitrary"),
        ),
    )(xt, xt, rel_pos,
      w1, s1, t1, a1, a2, s2, t2, g2, s3, t3, f1, s4, t4, f2, s5, t5)

    return jnp.transpose(out, (0, 2, 1)).reshape(_B, _C, _H, _W)


# speedup vs baseline: 566.4706x; 2.0862x over previous
import numpy as np
import jax
import jax.numpy as jnp
from jax.experimental import pallas as pl
from jax.experimental.pallas import tpu as pltpu

_C = 96
_K = 9
_G = 4
_EPS = 1e-05
_B, _H, _W = 16, 32, 32
_N = _H * _W
_R = 256
_PREC = jax.lax.Precision.HIGHEST
_PLOW = jax.lax.Precision.DEFAULT


def _gelu(x):
    return 0.5 * x * (1.0 + jax.lax.erf(x * np.float32(2.0 ** -0.5)))


def _body(xt_full_ref, xt_tile_ref, rp_ref,
          w1_ref, s1_ref, t1_ref,
          a1_ref, a2_ref, s2_ref, t2_ref,
          g2_ref, s3_ref, t3_ref,
          f1_ref, s4_ref, t4_ref,
          f2_ref, s5_ref, t5_ref,
          out_ref):
    xt = xt_full_ref[0]
    xt_t = xt_tile_ref[0]
    rp = rp_ref[0]

    h = jnp.dot(xt, w1_ref[...], preferred_element_type=jnp.float32,
                precision=_PREC) * s1_ref[...] + t1_ref[...]
    nrm = jnp.sqrt(jnp.sum(h * h, axis=1, keepdims=True))
    xn = h / jnp.maximum(nrm, 1e-12)
    xn2 = xn * xn
    ones_row = jnp.ones((1, _C), jnp.float32)
    sq_row = jax.lax.dot_general(ones_row, xn2, (((1,), (1,)), ((), ())),
                                 preferred_element_type=jnp.float32,
                                 precision=_PREC)

    h_t = jnp.dot(xt_t, w1_ref[...], preferred_element_type=jnp.float32,
                  precision=_PREC) * s1_ref[...] + t1_ref[...]
    nrm_t = jnp.sqrt(jnp.sum(h_t * h_t, axis=1, keepdims=True))
    xn_t = h_t / jnp.maximum(nrm_t, 1e-12)
    sq_t = jnp.sum(xn_t * xn_t, axis=1, keepdims=True)

    inner = jax.lax.dot_general(xn_t, xn, (((1,), (1,)), ((), ())),
                                preferred_element_type=jnp.float32,
                                precision=_PREC)
    d = sq_t + (-2.0) * inner + sq_row + rp

    col = jax.lax.broadcasted_iota(jnp.int32, (_R, _N), 1)
    maxg = jnp.full((_R, _C), -1e30, jnp.float32)
    for _ in range(_K):
        m = jnp.min(d, axis=1, keepdims=True)
        idx = jnp.min(jnp.where(d == m, col, _N), axis=1, keepdims=True)
        sel = col == idx
        onehot = sel.astype(jnp.float32)
        gk = jnp.dot(onehot, h, preferred_element_type=jnp.float32,
                     precision=_PLOW)
        maxg = jnp.maximum(maxg, gk)
        d = jnp.where(sel, 1e30, d)
    xjm = maxg - h_t

    mr = (jnp.dot(h_t, a1_ref[...], preferred_element_type=jnp.float32,
                  precision=_PLOW)
          + jnp.dot(xjm, a2_ref[...], preferred_element_type=jnp.float32,
                    precision=_PLOW))
    mr = _gelu(mr * s2_ref[...] + t2_ref[...])

    g = jnp.dot(mr, g2_ref[...], preferred_element_type=jnp.float32,
                precision=_PLOW) * s3_ref[...] + t3_ref[...]
    score = g + xt_t

    f = _gelu(jnp.dot(score, f1_ref[...], preferred_element_type=jnp.float32,
                      precision=_PLOW) * s4_ref[...] + t4_ref[...])
    f = jnp.dot(f, f2_ref[...], preferred_element_type=jnp.float32,
                precision=_PLOW) * s5_ref[...] + t5_ref[...]
    out_ref[0] = f + score


def kernel(x, g_fc1_w, g_fc1_b, g_fc1_gamma, g_fc1_beta, mr_w, mr_b,
           mr_gamma, mr_beta, g_fc2_w, g_fc2_b, g_fc2_gamma, g_fc2_beta,
           f_fc1_w, f_fc1_b, f_fc1_gamma, f_fc1_beta, f_fc2_w, f_fc2_b,
           f_fc2_gamma, f_fc2_beta, rel_pos):
    inv = np.float32(1.0 / np.sqrt(1.0 + _EPS))

    def fold(w_b, gamma, beta):
        s = gamma * inv
        return s[None, :], (w_b * s + beta)[None, :]

    s1, t1 = fold(g_fc1_b, g_fc1_gamma, g_fc1_beta)
    s2i, t2i = fold(mr_b, mr_gamma, mr_beta)
    s3, t3 = fold(g_fc2_b, g_fc2_gamma, g_fc2_beta)
    s4, t4 = fold(f_fc1_b, f_fc1_gamma, f_fc1_beta)
    s5, t5 = fold(f_fc2_b, f_fc2_gamma, f_fc2_beta)

    q = np.concatenate([2 * np.arange(_C), 2 * np.arange(_C) + 1])
    wg = mr_w.reshape(_G, 2 * _C // _G, 2 * _C // _G)
    w2 = jax.scipy.linalg.block_diag(*[wg[g] for g in range(_G)])
    a = w2[q][:, q]
    a1 = a[:, :_C].T
    a2 = a[:, _C:].T
    s2 = s2i[:, q]
    t2 = t2i[:, q]
    g2 = g_fc2_w[:, q].T

    w1 = g_fc1_w.T
    f1 = f_fc1_w.T
    f2 = f_fc2_w.T

    xt = jnp.transpose(x.reshape(_B, _C, _N), (0, 2, 1))

    nt = _N // _R
    grid = (nt, _B)

    def full_img(t, b):
        return (b, 0, 0)

    def row_tile(t, b):
        return (b, t, 0)

    def rp_tile(t, b):
        return (0, t, 0)

    def w_map(t, b):
        return (0, 0)

    wspec = lambda shape: pl.BlockSpec(shape, w_map)

    out = pl.pallas_call(
        _body,
        grid=grid,
        in_specs=[
            pl.BlockSpec((1, _N, _C), full_img),
            pl.BlockSpec((1, _R, _C), row_tile),
            pl.BlockSpec((1, _R, _N), rp_tile),
            wspec((_C, _C)),
            wspec((1, _C)), wspec((1, _C)),
            wspec((_C, 2 * _C)), wspec((_C, 2 * _C)),
            wspec((1, 2 * _C)), wspec((1, 2 * _C)),
            wspec((2 * _C, _C)),
            wspec((1, _C)), wspec((1, _C)),
            wspec((_C, 4 * _C)),
            wspec((1, 4 * _C)), wspec((1, 4 * _C)),
            wspec((4 * _C, _C)),
            wspec((1, _C)), wspec((1, _C)),
        ],
        out_specs=pl.BlockSpec((1, _R, _C), row_tile),
        out_shape=jax.ShapeDtypeStruct((_B, _N, _C), jnp.float32),
        compiler_params=pltpu.CompilerParams(
            dimension_semantics=("arbitrary", "arbitrary"),
        ),
    )(xt, xt, rel_pos,
      w1, s1, t1, a1, a2, s2, t2, g2, s3, t3, f1, s4, t4, f2, s5, t5)

    return jnp.transpose(out, (0, 2, 1)).reshape(_B, _C, _H, _W)


# speedup vs baseline: 953.0738x; 1.6825x over previous
import numpy as np
import jax
import jax.numpy as jnp
from jax.experimental import pallas as pl
from jax.experimental.pallas import tpu as pltpu

_C = 96
_K = 9
_G = 4
_EPS = 1e-05
_B, _H, _W = 16, 32, 32
_N = _H * _W
_R = 256
_PREC = jax.lax.Precision.HIGHEST
_PLOW = jax.lax.Precision.DEFAULT


def _gelu(x):
    return 0.5 * x * (1.0 + jax.lax.erf(x * np.float32(2.0 ** -0.5)))


def _feat_body(xt_ref, w1_ref, s1_ref, t1_ref, h_ref, xn_ref, sq_ref):
    xt = xt_ref[0]
    h = jnp.dot(xt, w1_ref[...], preferred_element_type=jnp.float32,
                precision=_PREC) * s1_ref[...] + t1_ref[...]
    nrm = jnp.sqrt(jnp.sum(h * h, axis=1, keepdims=True))
    xn = h / jnp.maximum(nrm, 1e-12)
    h_ref[0] = h
    xn_ref[0] = xn
    ones_row = jnp.ones((1, _C), jnp.float32)
    sq_ref[0] = jax.lax.dot_general(ones_row, xn * xn, (((1,), (1,)), ((), ())),
                                    preferred_element_type=jnp.float32,
                                    precision=_PREC)


def _body(h_full_ref, xn_full_ref, sq_row_ref, h_tile_ref, xn_tile_ref,
          xt_tile_ref, rp_ref,
          a1_ref, a2_ref, s2_ref, t2_ref,
          g2_ref, s3_ref, t3_ref,
          f1_ref, s4_ref, t4_ref,
          f2_ref, s5_ref, t5_ref,
          out_ref):
    h = h_full_ref[0]
    xn = xn_full_ref[0]
    sq_row = sq_row_ref[0]
    h_t = h_tile_ref[0]
    xn_t = xn_tile_ref[0]
    xt_t = xt_tile_ref[0]
    rp = rp_ref[0]

    sq_t = jnp.sum(xn_t * xn_t, axis=1, keepdims=True)
    inner = jax.lax.dot_general(xn_t, xn, (((1,), (1,)), ((), ())),
                                preferred_element_type=jnp.float32,
                                precision=_PREC)
    d = sq_t + (-2.0) * inner + sq_row + rp

    col = jax.lax.broadcasted_iota(jnp.int32, (_R, _N), 1)
    u = jax.lax.bitcast_convert_type(jnp.maximum(d + 16.0, 1.0), jnp.int32)
    keys = jax.lax.bitcast_convert_type((u & jnp.int32(-1024)) | col,
                                        jnp.float32)

    maxg = None
    for k in range(_K):
        kmin = jnp.min(keys, axis=1, keepdims=True)
        sel = keys == kmin
        onehot = sel.astype(jnp.float32)
        gk = jnp.dot(onehot, h, preferred_element_type=jnp.float32,
                     precision=_PLOW)
        maxg = gk if maxg is None else jnp.maximum(maxg, gk)
        if k < _K - 1:
            keys = jnp.where(sel, jnp.float32(3.0e38), keys)
    xjm = maxg - h_t

    mr = (jnp.dot(h_t, a1_ref[...], preferred_element_type=jnp.float32,
                  precision=_PLOW)
          + jnp.dot(xjm, a2_ref[...], preferred_element_type=jnp.float32,
                    precision=_PLOW))
    mr = _gelu(mr * s2_ref[...] + t2_ref[...])

    g = jnp.dot(mr, g2_ref[...], preferred_element_type=jnp.float32,
                precision=_PLOW) * s3_ref[...] + t3_ref[...]
    score = g + xt_t

    f = _gelu(jnp.dot(score, f1_ref[...], preferred_element_type=jnp.float32,
                      precision=_PLOW) * s4_ref[...] + t4_ref[...])
    f = jnp.dot(f, f2_ref[...], preferred_element_type=jnp.float32,
                precision=_PLOW) * s5_ref[...] + t5_ref[...]
    out_ref[0] = f + score


def kernel(x, g_fc1_w, g_fc1_b, g_fc1_gamma, g_fc1_beta, mr_w, mr_b,
           mr_gamma, mr_beta, g_fc2_w, g_fc2_b, g_fc2_gamma, g_fc2_beta,
           f_fc1_w, f_fc1_b, f_fc1_gamma, f_fc1_beta, f_fc2_w, f_fc2_b,
           f_fc2_gamma, f_fc2_beta, rel_pos):
    inv = np.float32(1.0 / np.sqrt(1.0 + _EPS))

    def fold(w_b, gamma, beta):
        s = gamma * inv
        return s[None, :], (w_b * s + beta)[None, :]

    s1, t1 = fold(g_fc1_b, g_fc1_gamma, g_fc1_beta)
    s2i, t2i = fold(mr_b, mr_gamma, mr_beta)
    s3, t3 = fold(g_fc2_b, g_fc2_gamma, g_fc2_beta)
    s4, t4 = fold(f_fc1_b, f_fc1_gamma, f_fc1_beta)
    s5, t5 = fold(f_fc2_b, f_fc2_gamma, f_fc2_beta)

    q = np.concatenate([2 * np.arange(_C), 2 * np.arange(_C) + 1])
    wg = mr_w.reshape(_G, 2 * _C // _G, 2 * _C // _G)
    w2 = jax.scipy.linalg.block_diag(*[wg[g] for g in range(_G)])
    a = w2[q][:, q]
    a1 = a[:, :_C].T
    a2 = a[:, _C:].T
    s2 = s2i[:, q]
    t2 = t2i[:, q]
    g2 = g_fc2_w[:, q].T

    w1 = g_fc1_w.T
    f1 = f_fc1_w.T
    f2 = f_fc2_w.T

    xt = jnp.transpose(x.reshape(_B, _C, _N), (0, 2, 1))

    h_all, xn_all, sq_all = pl.pallas_call(
        _feat_body,
        grid=(_B,),
        in_specs=[
            pl.BlockSpec((1, _N, _C), lambda b: (b, 0, 0)),
            pl.BlockSpec((_C, _C), lambda b: (0, 0)),
            pl.BlockSpec((1, _C), lambda b: (0, 0)),
            pl.BlockSpec((1, _C), lambda b: (0, 0)),
        ],
        out_specs=[
            pl.BlockSpec((1, _N, _C), lambda b: (b, 0, 0)),
            pl.BlockSpec((1, _N, _C), lambda b: (b, 0, 0)),
            pl.BlockSpec((1, 1, _N), lambda b: (b, 0, 0)),
        ],
        out_shape=[
            jax.ShapeDtypeStruct((_B, _N, _C), jnp.float32),
            jax.ShapeDtypeStruct((_B, _N, _C), jnp.float32),
            jax.ShapeDtypeStruct((_B, 1, _N), jnp.float32),
        ],
        compiler_params=pltpu.CompilerParams(
            dimension_semantics=("arbitrary",),
        ),
    )(xt, w1, s1, t1)

    nt = _N // _R
    grid = (nt, _B)

    def full_img(t, b):
        return (b, 0, 0)

    def row_tile(t, b):
        return (b, t, 0)

    def rp_tile(t, b):
        return (0, t, 0)

    def w_map(t, b):
        return (0, 0)

    wspec = lambda shape: pl.BlockSpec(shape, w_map)

    out = pl.pallas_call(
        _body,
        grid=grid,
        in_specs=[
            pl.BlockSpec((1, _N, _C), full_img),
            pl.BlockSpec((1, _N, _C), full_img),
            pl.BlockSpec((1, 1, _N), full_img),
            pl.BlockSpec((1, _R, _C), row_tile),
            pl.BlockSpec((1, _R, _C), row_tile),
            pl.BlockSpec((1, _R, _C), row_tile),
            pl.BlockSpec((1, _R, _N), rp_tile),
            wspec((_C, 2 * _C)), wspec((_C, 2 * _C)),
            wspec((1, 2 * _C)), wspec((1, 2 * _C)),
            wspec((2 * _C, _C)),
            wspec((1, _C)), wspec((1, _C)),
            wspec((_C, 4 * _C)),
            wspec((1, 4 * _C)), wspec((1, 4 * _C)),
            wspec((4 * _C, _C)),
            wspec((1, _C)), wspec((1, _C)),
        ],
        out_specs=pl.BlockSpec((1, _R, _C), row_tile),
        out_shape=jax.ShapeDtypeStruct((_B, _N, _C), jnp.float32),
        compiler_params=pltpu.CompilerParams(
            dimension_semantics=("arbitrary", "arbitrary"),
        ),
    )(h_all, xn_all, sq_all, h_all, xn_all, xt, rel_pos,
      a1, a2, s2, t2, g2, s3, t3, f1, s4, t4, f2, s5, t5)

    return jnp.transpose(out, (0, 2, 1)).reshape(_B, _C, _H, _W)
